# baseline (speedup 1.0000x reference)
import functools

import jax
import jax.numpy as jnp
from jax import lax
from jax.experimental import pallas as pl
from jax.experimental.pallas import tpu as pltpu
from jax.experimental.pallas import tpu_sc as plsc

N_NODES = 10000
D = 128
NP = 10240
NC = 2
NS = 16
NW = NC * NS
CH = 128
NCHUNK = 80
EPT = CH * NCHUNK
EPAD = EPT * NW
NCH0 = 160
NQ = 4
QC = NCH0 // NQ
PAD_NODE = N_NODES + 100
ACC_R = NP
ZPT = ACC_R // NS
RPT = ACC_R // NS

_mesh = plsc.VectorSubcoreMesh(
    core_axis_name="c", subcore_axis_name="s", num_cores=NC, num_subcores=NS)


@functools.partial(
    pl.kernel,
    out_type=(jax.ShapeDtypeStruct((NW, NP), jnp.float32),
              jax.ShapeDtypeStruct((NW, NP), jnp.float32)),
    mesh=_mesh,
    scratch_types=[
        pltpu.VMEM((EPT,), jnp.int32),
        pltpu.VMEM((EPT,), jnp.int32),
        pltpu.VMEM((NP,), jnp.float32),
        pltpu.VMEM((NP,), jnp.float32),
    ],
    compiler_params=pltpu.CompilerParams(needs_layout_passes=False),
)
def _deg_kernel(src_hbm, dst_hbm, osrc_hbm, odst_hbm, src_v, dst_v, dsrc, ddst):
    c = lax.axis_index("c")
    s = lax.axis_index("s")
    w = s * NC + c
    pltpu.sync_copy(src_hbm.at[w], src_v)
    pltpu.sync_copy(dst_hbm.at[w], dst_v)

    zero16 = jnp.zeros((16,), jnp.float32)

    def zbody(i, _):
        dsrc[pl.ds(i * 16, 16)] = zero16
        ddst[pl.ds(i * 16, 16)] = zero16
        return 0

    lax.fori_loop(0, NP // 16, zbody, 0)

    ones16 = jnp.ones((16,), jnp.float32)

    def ebody(i, _):
        si = src_v[pl.ds(i * 16, 16)]
        plsc.addupdate_scatter(dsrc, [si], ones16)
        di = dst_v[pl.ds(i * 16, 16)]
        plsc.addupdate_scatter(ddst, [di], ones16)
        return 0

    lax.fori_loop(0, EPT // 16, ebody, 0)

    pltpu.sync_copy(dsrc, osrc_hbm.at[w])
    pltpu.sync_copy(ddst, odst_hbm.at[w])


@functools.partial(
    pl.kernel,
    out_type=jax.ShapeDtypeStruct((NP, D), jnp.float32),
    mesh=_mesh,
    scratch_types=[
        pltpu.VMEM((QC, CH), jnp.int32),
        pltpu.VMEM((QC, CH), jnp.int32),
        pltpu.VMEM((CH, D), jnp.float32),
        pltpu.VMEM((CH, D), jnp.float32),
        pltpu.VMEM_SHARED((ACC_R, D), jnp.float32),
        pltpu.SemaphoreType.DMA,
        pltpu.SemaphoreType.DMA,
    ],
)
def _msg_kernel(h_hbm, src_hbm, dst_hbm, zero_hbm, out_hbm,
                src_v, dst_v, buf0, buf1, acc, sem0, sem1):
    c = lax.axis_index("c")
    s = lax.axis_index("s")

    @pl.when(c == 0)
    def _():
        pltpu.sync_copy(zero_hbm, acc.at[pl.ds(s * ZPT, ZPT)])
        plsc.subcore_barrier()

        npairs = NCH0 // NQ // 2

        for q in range(NQ):
            pltpu.sync_copy(src_hbm.at[s, q], src_v)
            pltpu.sync_copy(dst_hbm.at[s, q], dst_v)
            pltpu.async_copy(h_hbm.at[src_v.at[0]], buf0, sem0)

            def body(jj, _):
                j0 = 2 * jj
                pltpu.async_copy(h_hbm.at[src_v.at[j0 + 1]], buf1, sem1)
                pltpu.make_async_copy(h_hbm.at[src_v.at[j0]], buf0, sem0).wait()
                pltpu.sync_copy(buf0, acc.at[dst_v.at[j0]], add=True)

                @pl.when(jj < npairs - 1)
                def _():
                    pltpu.async_copy(h_hbm.at[src_v.at[j0 + 2]], buf0, sem0)

                pltpu.make_async_copy(h_hbm.at[src_v.at[j0 + 1]], buf1, sem1).wait()
                pltpu.sync_copy(buf1, acc.at[dst_v.at[j0 + 1]], add=True)
                return 0

            lax.fori_loop(0, npairs, body, 0)

        plsc.subcore_barrier()
        pltpu.sync_copy(acc.at[pl.ds(s * RPT, RPT)], out_hbm.at[pl.ds(s * RPT, RPT)])


def _norm_body(s_ref, d_ref, ns_ref, nd_ref):
    sdeg = jnp.sum(s_ref[...], axis=0, keepdims=True)
    ddeg = jnp.sum(d_ref[...], axis=0, keepdims=True)
    ns_ref[...] = lax.rsqrt(jnp.clip(sdeg, 1.0, None))
    nd_ref[...] = lax.rsqrt(jnp.clip(ddeg, 1.0, None))


_norm_tc = pl.pallas_call(
    _norm_body,
    out_shape=(jax.ShapeDtypeStruct((1, NP), jnp.float32),
               jax.ShapeDtypeStruct((1, NP), jnp.float32)),
)


def _scale_body(x_ref, n_ref, o_ref):
    o_ref[...] = x_ref[...] * n_ref[...]


_BLK = 1024


def _scale_tc(x, n_col):
    return pl.pallas_call(
        _scale_body,
        grid=(NP // _BLK,),
        in_specs=[pl.BlockSpec((_BLK, D), lambda i: (i, 0)),
                  pl.BlockSpec((_BLK, 1), lambda i: (i, 0))],
        out_specs=pl.BlockSpec((_BLK, D), lambda i: (i, 0)),
        out_shape=jax.ShapeDtypeStruct((NP, D), jnp.float32),
    )(x, n_col)


def _mid_body(acc_ref, nd_ref, ns_ref, w_ref, b_ref, o_ref):
    agg = acc_ref[...] * nd_ref[...]
    y = jnp.dot(agg, w_ref[...], preferred_element_type=jnp.float32) + b_ref[...]
    o_ref[...] = jax.nn.relu(y) * ns_ref[...]


_MBLK = 1024


def _mid_tc(acc, nd_col, ns_col, w, b_row):
    return pl.pallas_call(
        _mid_body,
        grid=(NP // _MBLK,),
        in_specs=[pl.BlockSpec((_MBLK, D), lambda i: (i, 0)),
                  pl.BlockSpec((_MBLK, 1), lambda i: (i, 0)),
                  pl.BlockSpec((_MBLK, 1), lambda i: (i, 0)),
                  pl.BlockSpec((D, D), lambda i: (0, 0)),
                  pl.BlockSpec((1, D), lambda i: (0, 0))],
        out_specs=pl.BlockSpec((_MBLK, D), lambda i: (i, 0)),
        out_shape=jax.ShapeDtypeStruct((NP, D), jnp.float32),
    )(acc, nd_col, ns_col, w, b_row)


def _fin_body(acc_ref, nd_ref, w_ref, b_ref, o_ref):
    agg = acc_ref[...] * nd_ref[...]
    o_ref[...] = jnp.dot(agg, w_ref[...], preferred_element_type=jnp.float32) + b_ref[...]


def _fin_tc(acc, nd_col, w, b_row):
    return pl.pallas_call(
        _fin_body,
        grid=(NP // _MBLK,),
        in_specs=[pl.BlockSpec((_MBLK, D), lambda i: (i, 0)),
                  pl.BlockSpec((_MBLK, 1), lambda i: (i, 0)),
                  pl.BlockSpec((D, D), lambda i: (0, 0)),
                  pl.BlockSpec((1, D), lambda i: (0, 0))],
        out_specs=pl.BlockSpec((_MBLK, D), lambda i: (i, 0)),
        out_shape=jax.ShapeDtypeStruct((NP, D), jnp.float32),
    )(acc, nd_col, w, b_row)


def kernel(features, edge_index, W1, b1, W2, b2):
    feat = features.astype(jnp.float32)
    ei = edge_index.astype(jnp.int32)
    n_edges = ei.shape[1]
    pad = EPAD - n_edges
    src_deg = jnp.concatenate(
        [ei[0], jnp.full((pad,), PAD_NODE, jnp.int32)]).reshape(NW, EPT)
    dst_deg = jnp.concatenate(
        [ei[1], jnp.full((pad,), PAD_NODE, jnp.int32)]).reshape(NW, EPT)
    dst_msg = jnp.concatenate([ei[1], jnp.zeros((pad,), jnp.int32)])

    src3 = src_deg.reshape(NS, NQ, QC, CH)
    dst3 = dst_msg.reshape(NS, NQ, QC, CH)

    feat_p = jnp.pad(feat, ((0, NP - N_NODES), (0, 0)))
    zrows = jnp.zeros((ZPT, D), jnp.float32)
    b1r = b1.reshape(1, D)
    b2r = b2.reshape(1, D)

    dsrc_p, ddst_p = _deg_kernel(src_deg, dst_deg)
    ns_row, nd_row = _norm_tc(dsrc_p, ddst_p)
    ns_col = ns_row.reshape(NP, 1)
    nd_col = nd_row.reshape(NP, 1)

    h1 = _scale_tc(feat_p, ns_col)
    acc1 = _msg_kernel(h1, src3, dst3, zrows)
    h2 = _mid_tc(acc1, nd_col, ns_col, W1, b1r)
    acc2 = _msg_kernel(h2, src3, dst3, zrows)
    return _fin_tc(acc2, nd_col, W2, b2r)[:N_NODES]


# speedup vs baseline: 1.6090x; 1.6090x over previous
import functools

import jax
import jax.numpy as jnp
from jax import lax
from jax.experimental import pallas as pl
from jax.experimental.pallas import tpu as pltpu
from jax.experimental.pallas import tpu_sc as plsc

N_NODES = 10000
D = 128
NP = 10240
NC = 2
NS = 16
NW = NC * NS
CH = 128
NCHUNK = 80
EPT = CH * NCHUNK
EPAD = EPT * NW
NCH0 = 144
NCH1 = 16
NQ = 3
QC = NCH0 // NQ
PAD_NODE = N_NODES + 100
ACC_R = NP
ZPT = ACC_R // NS
RPT = ACC_R // NS

_mesh = plsc.VectorSubcoreMesh(
    core_axis_name="c", subcore_axis_name="s", num_cores=NC, num_subcores=NS)


@functools.partial(
    pl.kernel,
    out_type=(jax.ShapeDtypeStruct((NW, NP), jnp.float32),
              jax.ShapeDtypeStruct((NW, NP), jnp.float32)),
    mesh=_mesh,
    scratch_types=[
        pltpu.VMEM((EPT,), jnp.int32),
        pltpu.VMEM((EPT,), jnp.int32),
        pltpu.VMEM((NP,), jnp.float32),
        pltpu.VMEM((NP,), jnp.float32),
    ],
    compiler_params=pltpu.CompilerParams(needs_layout_passes=False),
)
def _deg_kernel(src_hbm, dst_hbm, osrc_hbm, odst_hbm, src_v, dst_v, dsrc, ddst):
    c = lax.axis_index("c")
    s = lax.axis_index("s")
    w = s * NC + c
    pltpu.sync_copy(src_hbm.at[w], src_v)
    pltpu.sync_copy(dst_hbm.at[w], dst_v)

    zero16 = jnp.zeros((16,), jnp.float32)

    def zbody(i, _):
        dsrc[pl.ds(i * 16, 16)] = zero16
        ddst[pl.ds(i * 16, 16)] = zero16
        return 0

    lax.fori_loop(0, NP // 16, zbody, 0)

    ones16 = jnp.ones((16,), jnp.float32)

    def ebody(i, _):
        si = src_v[pl.ds(i * 16, 16)]
        plsc.addupdate_scatter(dsrc, [si], ones16)
        di = dst_v[pl.ds(i * 16, 16)]
        plsc.addupdate_scatter(ddst, [di], ones16)
        return 0

    lax.fori_loop(0, EPT // 16, ebody, 0)

    pltpu.sync_copy(dsrc, osrc_hbm.at[w])
    pltpu.sync_copy(ddst, odst_hbm.at[w])


@functools.partial(
    pl.kernel,
    out_type=jax.ShapeDtypeStruct((NC, NP, D), jnp.float32),
    mesh=_mesh,
    scratch_types=[
        pltpu.VMEM((QC, CH), jnp.int32),
        pltpu.VMEM((QC, CH), jnp.int32),
        pltpu.VMEM((CH, D), jnp.float32),
        pltpu.VMEM((CH, D), jnp.float32),
        pltpu.VMEM_SHARED((ACC_R, D), jnp.float32),
        pltpu.SemaphoreType.DMA,
        pltpu.SemaphoreType.DMA,
    ],
)
def _msg_kernel(h_hbm, src_hbm, dst_hbm, zero_hbm, out_hbm,
                src_v, dst_v, buf0, buf1, acc, sem0, sem1):
    c = lax.axis_index("c")
    s = lax.axis_index("s")
    pltpu.sync_copy(zero_hbm, acc.at[pl.ds(s * ZPT, ZPT)])
    plsc.subcore_barrier()

    for q in range(NQ):
        npairs = jnp.where(c == 0, NCH0 // NQ // 2,
                           NCH1 // 2 if q == 0 else 0)
        pltpu.sync_copy(src_hbm.at[c, s, q], src_v)
        pltpu.sync_copy(dst_hbm.at[c, s, q], dst_v)

        @pl.when(npairs > 0)
        def _():
            pltpu.async_copy(h_hbm.at[src_v.at[0]], buf0, sem0)

        def body(jj, _):
            j0 = 2 * jj
            pltpu.async_copy(h_hbm.at[src_v.at[j0 + 1]], buf1, sem1)
            pltpu.make_async_copy(h_hbm.at[src_v.at[j0]], buf0, sem0).wait()
            pltpu.sync_copy(buf0, acc.at[dst_v.at[j0]], add=True)

            @pl.when(jj < npairs - 1)
            def _():
                pltpu.async_copy(h_hbm.at[src_v.at[j0 + 2]], buf0, sem0)

            pltpu.make_async_copy(h_hbm.at[src_v.at[j0 + 1]], buf1, sem1).wait()
            pltpu.sync_copy(buf1, acc.at[dst_v.at[j0 + 1]], add=True)
            return 0

        lax.fori_loop(0, npairs, body, 0)

    plsc.subcore_barrier()
    pltpu.sync_copy(acc.at[pl.ds(s * RPT, RPT)], out_hbm.at[c, pl.ds(s * RPT, RPT)])


def _norm_body(s_ref, d_ref, ns_ref, nd_ref):
    sdeg = jnp.sum(s_ref[...], axis=0, keepdims=True)
    ddeg = jnp.sum(d_ref[...], axis=0, keepdims=True)
    ns_ref[...] = lax.rsqrt(jnp.clip(sdeg, 1.0, None))
    nd_ref[...] = lax.rsqrt(jnp.clip(ddeg, 1.0, None))


_norm_tc = pl.pallas_call(
    _norm_body,
    out_shape=(jax.ShapeDtypeStruct((1, NP), jnp.float32),
               jax.ShapeDtypeStruct((1, NP), jnp.float32)),
)


def _scale_body(x_ref, n_ref, o_ref):
    o_ref[...] = x_ref[...] * n_ref[...]


_BLK = 1024


def _scale_tc(x, n_col):
    return pl.pallas_call(
        _scale_body,
        grid=(NP // _BLK,),
        in_specs=[pl.BlockSpec((_BLK, D), lambda i: (i, 0)),
                  pl.BlockSpec((_BLK, 1), lambda i: (i, 0))],
        out_specs=pl.BlockSpec((_BLK, D), lambda i: (i, 0)),
        out_shape=jax.ShapeDtypeStruct((NP, D), jnp.float32),
    )(x, n_col)


def _mid_body(acc_ref, nd_ref, ns_ref, w_ref, b_ref, o_ref):
    agg = (acc_ref[0] + acc_ref[1]) * nd_ref[...]
    y = jnp.dot(agg, w_ref[...], preferred_element_type=jnp.float32) + b_ref[...]
    o_ref[...] = jax.nn.relu(y) * ns_ref[...]


_MBLK = 1024


def _mid_tc(acc, nd_col, ns_col, w, b_row):
    return pl.pallas_call(
        _mid_body,
        grid=(NP // _MBLK,),
        in_specs=[pl.BlockSpec((NC, _MBLK, D), lambda i: (0, i, 0)),
                  pl.BlockSpec((_MBLK, 1), lambda i: (i, 0)),
                  pl.BlockSpec((_MBLK, 1), lambda i: (i, 0)),
                  pl.BlockSpec((D, D), lambda i: (0, 0)),
                  pl.BlockSpec((1, D), lambda i: (0, 0))],
        out_specs=pl.BlockSpec((_MBLK, D), lambda i: (i, 0)),
        out_shape=jax.ShapeDtypeStruct((NP, D), jnp.float32),
    )(acc, nd_col, ns_col, w, b_row)


def _fin_body(acc_ref, nd_ref, w_ref, b_ref, o_ref):
    agg = (acc_ref[0] + acc_ref[1]) * nd_ref[...]
    o_ref[...] = jnp.dot(agg, w_ref[...], preferred_element_type=jnp.float32) + b_ref[...]


def _fin_tc(acc, nd_col, w, b_row):
    return pl.pallas_call(
        _fin_body,
        grid=(NP // _MBLK,),
        in_specs=[pl.BlockSpec((NC, _MBLK, D), lambda i: (0, i, 0)),
                  pl.BlockSpec((_MBLK, 1), lambda i: (i, 0)),
                  pl.BlockSpec((D, D), lambda i: (0, 0)),
                  pl.BlockSpec((1, D), lambda i: (0, 0))],
        out_specs=pl.BlockSpec((_MBLK, D), lambda i: (i, 0)),
        out_shape=jax.ShapeDtypeStruct((NP, D), jnp.float32),
    )(acc, nd_col, w, b_row)


def kernel(features, edge_index, W1, b1, W2, b2):
    feat = features.astype(jnp.float32)
    ei = edge_index.astype(jnp.int32)
    n_edges = ei.shape[1]
    pad = EPAD - n_edges
    src_deg = jnp.concatenate(
        [ei[0], jnp.full((pad,), PAD_NODE, jnp.int32)]).reshape(NW, EPT)
    dst_deg = jnp.concatenate(
        [ei[1], jnp.full((pad,), PAD_NODE, jnp.int32)]).reshape(NW, EPT)
    dst_msg = jnp.concatenate([ei[1], jnp.zeros((pad,), jnp.int32)])

    def _split(flat):
        f = flat.reshape(EPAD // CH, CH)
        n0 = NS * NCH0
        a0 = f[:n0].reshape(NS, NQ, QC, CH)
        a1 = f[n0:].reshape(NS, 1, NCH1, CH)
        a1 = jnp.pad(a1, ((0, 0), (0, NQ - 1), (0, QC - NCH1), (0, 0)))
        return jnp.stack([a0, a1])

    src3 = _split(src_deg.reshape(EPAD))
    dst3 = _split(dst_msg)

    feat_p = jnp.pad(feat, ((0, NP - N_NODES), (0, 0)))
    zrows = jnp.zeros((ZPT, D), jnp.float32)
    b1r = b1.reshape(1, D)
    b2r = b2.reshape(1, D)

    dsrc_p, ddst_p = _deg_kernel(src_deg, dst_deg)
    ns_row, nd_row = _norm_tc(dsrc_p, ddst_p)
    ns_col = ns_row.reshape(NP, 1)
    nd_col = nd_row.reshape(NP, 1)

    h1 = _scale_tc(feat_p, ns_col)
    acc1 = _msg_kernel(h1, src3, dst3, zrows)
    h2 = _mid_tc(acc1, nd_col, ns_col, W1, b1r)
    acc2 = _msg_kernel(h2, src3, dst3, zrows)
    return _fin_tc(acc2, nd_col, W2, b2r)[:N_NODES]
